# baseline (speedup 1.0000x reference)
import functools

import jax
import jax.numpy as jnp
from jax import lax
from jax.experimental import pallas as pl
from jax.experimental.pallas import tpu as pltpu
from jax.experimental.pallas import tpu_sc as plsc

STORAGE_SIZE = 100000
FEATURES_SIZE = 64
BATCH = 4096

_info = plsc.get_sparse_core_info()
_NC, _NS = _info.num_cores, _info.num_subcores
_NW = _NC * _NS
_BPW = BATCH // _NW
_L = 16

_mesh = plsc.VectorSubcoreMesh(core_axis_name="c", subcore_axis_name="s")


@functools.partial(
    pl.kernel,
    mesh=_mesh,
    out_type=jax.ShapeDtypeStruct((BATCH, FEATURES_SIZE), jnp.float32),
    scratch_types=[
        pltpu.VMEM((_BPW,), jnp.int32),
        pltpu.VMEM((_BPW, FEATURES_SIZE), jnp.float32),
        pltpu.SemaphoreType.DMA,
    ],
)
def _gather_kernel(idx_hbm, tab4_hbm, out_hbm, idx_v, rows_v, sem):
    wid = lax.axis_index("s") * _NC + lax.axis_index("c")
    base = wid * _BPW
    pltpu.sync_copy(idx_hbm.at[pl.ds(base, _BPW)], idx_v)

    def issue(c, carry):
        rv = idx_v[pl.ds(c * _L, _L)]
        for j in range(_L):
            pltpu.async_copy(tab4_hbm.at[rv[j]],
                             rows_v.at[c * _L + j], sem)
        return carry

    lax.fori_loop(0, _BPW // _L, issue, 0)
    pltpu.make_async_copy(out_hbm.at[pl.ds(base, _BPW)], rows_v, sem).wait()
    pltpu.sync_copy(rows_v, out_hbm.at[pl.ds(base, _BPW)])


@jax.jit
def kernel(indexes, table):
    return _gather_kernel(indexes.astype(jnp.int32), table)


# speedup vs baseline: 1.1861x; 1.1861x over previous
import functools

import jax
import jax.numpy as jnp
from jax import lax
from jax.experimental import pallas as pl
from jax.experimental.pallas import tpu as pltpu
from jax.experimental.pallas import tpu_sc as plsc

STORAGE_SIZE = 100000
FEATURES_SIZE = 64
BATCH = 4096

_info = plsc.get_sparse_core_info()
_NC, _NS = _info.num_cores, _info.num_subcores
_NW = _NC * _NS
_BPW = BATCH // _NW
_L = 16

_mesh = plsc.VectorSubcoreMesh(core_axis_name="c", subcore_axis_name="s")


@functools.partial(
    pl.kernel,
    mesh=_mesh,
    out_type=jax.ShapeDtypeStruct((BATCH, 2 * FEATURES_SIZE), jnp.float32),
    scratch_types=[
        pltpu.VMEM((_BPW,), jnp.int32),
        pltpu.VMEM((_BPW, FEATURES_SIZE), jnp.float32),
        pltpu.VMEM((_BPW, 2 * FEATURES_SIZE), jnp.float32),
        pltpu.SemaphoreType.DMA,
    ],
)
def _gather_kernel(idx_hbm, tab4_hbm, out_hbm, idx_v, rows_v, wide_v, sem):
    wid = lax.axis_index("s") * _NC + lax.axis_index("c")
    base = wid * _BPW
    pltpu.sync_copy(idx_hbm.at[pl.ds(base, _BPW)], idx_v)

    def issue(c, carry):
        rv = idx_v[pl.ds(c * _L, _L)]
        tv = lax.shift_right_logical(rv, 3)
        sv = lax.rem(rv, 8)
        for j in range(_L):
            pltpu.async_copy(tab4_hbm.at[tv[j], sv[j]],
                             rows_v.at[c * _L + j], sem)
        return carry

    lax.fori_loop(0, _BPW // _L, issue, 0)
    pltpu.make_async_copy(out_hbm.at[pl.ds(base, _BPW // 2)], rows_v,
                          sem).wait()

    def widen(c, carry):
        for j in range(_L):
            i = c * _L + j
            for g in range(FEATURES_SIZE // _L):
                wide_v[i, pl.ds(g * _L, _L)] = rows_v[i, pl.ds(g * _L, _L)]
        return carry

    lax.fori_loop(0, _BPW // _L, widen, 0)
    pltpu.sync_copy(wide_v, out_hbm.at[pl.ds(base, _BPW)])


@jax.jit
def kernel(indexes, table):
    tab4 = table.reshape(STORAGE_SIZE // 8, 8, FEATURES_SIZE)
    padded = _gather_kernel(indexes.astype(jnp.int32), tab4)
    return padded[:, :FEATURES_SIZE]


# speedup vs baseline: 1.2059x; 1.0167x over previous
import functools

import jax
import jax.numpy as jnp
from jax import lax
from jax.experimental import pallas as pl
from jax.experimental.pallas import tpu as pltpu
from jax.experimental.pallas import tpu_sc as plsc

STORAGE_SIZE = 100000
FEATURES_SIZE = 64
BATCH = 4096

_info = plsc.get_sparse_core_info()
_NC, _NS = _info.num_cores, _info.num_subcores
_NW = _NC * _NS
_BPW = BATCH // _NW
_L = 16

_mesh = plsc.VectorSubcoreMesh(core_axis_name="c", subcore_axis_name="s")


@functools.partial(
    pl.kernel,
    mesh=_mesh,
    out_type=jax.ShapeDtypeStruct((BATCH // 8, 8, FEATURES_SIZE), jnp.float32),
    scratch_types=[
        pltpu.VMEM((_BPW,), jnp.int32),
        pltpu.VMEM((_BPW // 8, 8, FEATURES_SIZE), jnp.float32),
        pltpu.SemaphoreType.DMA,
    ],
)
def _gather_kernel(idx_hbm, tab4_hbm, out_hbm, idx_v, rows_v, sem):
    wid = lax.axis_index("s") * _NC + lax.axis_index("c")
    base = wid * _BPW
    pltpu.sync_copy(idx_hbm.at[pl.ds(base, _BPW)], idx_v)

    def issue(c, carry):
        rv = idx_v[pl.ds(c * _L, _L)]
        tv = lax.shift_right_logical(rv, 3)
        sv = lax.rem(rv, 8)
        for j in range(_L):
            pltpu.async_copy(tab4_hbm.at[tv[j], sv[j]],
                             rows_v.at[c * 2 + j // 8, j % 8], sem)
        return carry

    lax.fori_loop(0, _BPW // _L, issue, 0)
    pltpu.make_async_copy(out_hbm.at[pl.ds(base // 8, _BPW // 8)], rows_v,
                          sem).wait()
    pltpu.sync_copy(rows_v, out_hbm.at[pl.ds(base // 8, _BPW // 8)])


@jax.jit
def kernel(indexes, table):
    tab4 = table.reshape(STORAGE_SIZE // 8, 8, FEATURES_SIZE)
    out3 = _gather_kernel(indexes.astype(jnp.int32), tab4)
    return out3.reshape(BATCH, FEATURES_SIZE)


# speedup vs baseline: 1.2102x; 1.0035x over previous
import functools

import jax
import jax.numpy as jnp
from jax import lax
from jax.experimental import pallas as pl
from jax.experimental.pallas import tpu as pltpu
from jax.experimental.pallas import tpu_sc as plsc

STORAGE_SIZE = 100000
FEATURES_SIZE = 64
BATCH = 4096

_info = plsc.get_sparse_core_info()
_NC, _NS = _info.num_cores, _info.num_subcores
_NW = _NC * _NS
_BPW = BATCH // _NW
_L = 16

_mesh = plsc.VectorSubcoreMesh(core_axis_name="c", subcore_axis_name="s")


@functools.partial(
    pl.kernel,
    mesh=_mesh,
    out_type=jax.ShapeDtypeStruct((BATCH, FEATURES_SIZE), jnp.float32),
    scratch_types=[
        pltpu.VMEM((_BPW,), jnp.int32),
        pltpu.VMEM((_BPW, FEATURES_SIZE), jnp.float32),
        pltpu.SemaphoreType.DMA,
    ],
)
def _gather_kernel(idx_hbm, tab4_hbm, out_hbm, idx_v, rows_v, sem):
    wid = lax.axis_index("s") * _NC + lax.axis_index("c")
    base = wid * _BPW
    pltpu.sync_copy(idx_hbm.at[pl.ds(base, _BPW)], idx_v)

    def issue(c, carry):
        rv = idx_v[pl.ds(c * _L, _L)]
        tv = lax.shift_right_logical(rv, 3)
        sv = lax.rem(rv, 8)
        for j in range(_L):
            pltpu.async_copy(tab4_hbm.at[tv[j], sv[j]],
                             rows_v.at[c * _L + j], sem)
        return carry

    lax.fori_loop(0, _BPW // _L, issue, 0)
    pltpu.make_async_copy(out_hbm.at[pl.ds(base, _BPW)], rows_v, sem).wait()
    pltpu.sync_copy(rows_v, out_hbm.at[pl.ds(base, _BPW)])


@jax.jit
def kernel(indexes, table):
    tab4 = table.reshape(STORAGE_SIZE // 8, 8, FEATURES_SIZE)
    return _gather_kernel(indexes.astype(jnp.int32), tab4)
